# baseline (speedup 1.0000x reference)
import functools

import jax
import jax.numpy as jnp
from jax import lax
from jax.experimental import pallas as pl
from jax.experimental.pallas import tpu as pltpu
from jax.experimental.pallas import tpu_sc as plsc

SEQ = 2048
HIDDEN = 768
LANES = 16
NCORES = 2
NSUB = 16
NW = NCORES * NSUB
RPW = SEQ // NW
NCHUNK = 4
CROWS = RPW // NCHUNK
NPASS = 3
PCH = HIDDEN // (NPASS * LANES)

_MESH = plsc.VectorSubcoreMesh(core_axis_name="c", subcore_axis_name="s")


@functools.partial(
    pl.kernel,
    out_type=jax.ShapeDtypeStruct((NW, HIDDEN), jnp.float32),
    mesh=_MESH,
    scratch_types=[
        pltpu.VMEM((RPW,), jnp.int32),
        pltpu.VMEM((RPW, HIDDEN), jnp.float32),
        pltpu.VMEM((1, HIDDEN), jnp.float32),
        [pltpu.SemaphoreType.DMA] * NCHUNK,
    ],
)
def _sc_partials(ids_hbm, wt_hbm, out_hbm, idx_v, rows_v, acc_v, sems):
    wid = lax.axis_index("s") * NCORES + lax.axis_index("c")
    base = wid * RPW
    pltpu.sync_copy(ids_hbm.at[pl.ds(base, RPW)], idx_v)
    gathers = []
    for c in range(NCHUNK):
        gathers.append(pltpu.async_copy(
            wt_hbm.at[idx_v.at[pl.ds(c * CROWS, CROWS)]],
            rows_v.at[pl.ds(c * CROWS, CROWS)],
            sems[c]))

    zero = jnp.zeros((LANES,), jnp.float32)
    UNROLL = 4
    for c in range(NCHUNK):
        gathers[c].wait()
        for p in range(NPASS):
            def body(r, carry):
                for u in range(UNROLL):
                    carry = tuple(
                        carry[j] + rows_v[c * CROWS + r * UNROLL + u,
                                          pl.ds((p * PCH + j) * LANES, LANES)]
                        for j in range(PCH))
                return carry
            carry = lax.fori_loop(0, CROWS // UNROLL, body, (zero,) * PCH)
            for j in range(PCH):
                sl = pl.ds((p * PCH + j) * LANES, LANES)
                if c == 0:
                    acc_v[0, sl] = carry[j]
                else:
                    plsc.addupdate(acc_v.at[0, sl], carry[j])

    pltpu.sync_copy(acc_v, out_hbm.at[pl.ds(wid, 1)])


POS_BLK = 256


def _pos_sum_body(pos_ref, o_ref):
    step = pl.program_id(0)

    @pl.when(step == 0)
    def _init():
        o_ref[...] = jnp.zeros_like(o_ref)

    o_ref[...] += jnp.sum(pos_ref[...], axis=0, keepdims=True)


_pos_sum = pl.pallas_call(
    _pos_sum_body,
    grid=(SEQ // POS_BLK,),
    in_specs=[pl.BlockSpec((POS_BLK, HIDDEN), lambda i: (i, 0))],
    out_specs=pl.BlockSpec((1, HIDDEN), lambda i: (0, 0)),
    out_shape=jax.ShapeDtypeStruct((1, HIDDEN), jnp.float32),
)


def _final_body(parts_ref, pos_ref, o_ref):
    o_ref[...] = (jnp.sum(parts_ref[...], axis=0)
                  + pos_ref[0, :]) * (1.0 / SEQ)


_final = pl.pallas_call(
    _final_body,
    out_shape=jax.ShapeDtypeStruct((HIDDEN,), jnp.float32),
)


def kernel(input_ids, word_table, pos_table):
    ids = input_ids.reshape(-1).astype(jnp.int32)
    parts = _sc_partials(ids, word_table)
    pos_part = _pos_sum(pos_table)
    return _final(parts, pos_part)


# speedup vs baseline: 1.0725x; 1.0725x over previous
import functools

import jax
import jax.numpy as jnp
from jax import lax
from jax.experimental import pallas as pl
from jax.experimental.pallas import tpu as pltpu
from jax.experimental.pallas import tpu_sc as plsc

SEQ = 2048
HIDDEN = 768
LANES = 16
NCORES = 2
NSUB = 16
NW = NCORES * NSUB
RPW = SEQ // NW
NCHUNK = 4
CROWS = RPW // NCHUNK
NPASS = 3
PCH = HIDDEN // (NPASS * LANES)

_MESH = plsc.VectorSubcoreMesh(core_axis_name="c", subcore_axis_name="s")


@functools.partial(
    pl.kernel,
    out_type=jax.ShapeDtypeStruct((NW, HIDDEN), jnp.float32),
    mesh=_MESH,
    scratch_types=[
        pltpu.VMEM((RPW,), jnp.int32),
        pltpu.VMEM((RPW, HIDDEN), jnp.float32),
        pltpu.VMEM((1, HIDDEN), jnp.float32),
        [pltpu.SemaphoreType.DMA] * NCHUNK,
    ],
)
def _sc_partials(ids_hbm, wt_hbm, out_hbm, idx_v, rows_v, acc_v, sems):
    wid = lax.axis_index("s") * NCORES + lax.axis_index("c")
    base = wid * RPW
    pltpu.sync_copy(ids_hbm.at[pl.ds(base, RPW)], idx_v)
    gathers = []
    for c in range(NCHUNK):
        gathers.append(pltpu.async_copy(
            wt_hbm.at[idx_v.at[pl.ds(c * CROWS, CROWS)]],
            rows_v.at[pl.ds(c * CROWS, CROWS)],
            sems[c]))

    zero = jnp.zeros((LANES,), jnp.float32)
    UNROLL = 1
    for c in range(NCHUNK):
        gathers[c].wait()
        for p in range(NPASS):
            def body(r, carry):
                for u in range(UNROLL):
                    carry = tuple(
                        carry[j] + rows_v[c * CROWS + r * UNROLL + u,
                                          pl.ds((p * PCH + j) * LANES, LANES)]
                        for j in range(PCH))
                return carry
            carry = lax.fori_loop(0, CROWS // UNROLL, body, (zero,) * PCH)
            for j in range(PCH):
                sl = pl.ds((p * PCH + j) * LANES, LANES)
                if c == 0:
                    acc_v[0, sl] = carry[j]
                else:
                    plsc.addupdate(acc_v.at[0, sl], carry[j])

    pltpu.sync_copy(acc_v, out_hbm.at[pl.ds(wid, 1)])


POS_BLK = 256


def _pos_sum_body(pos_ref, o_ref):
    step = pl.program_id(0)

    @pl.when(step == 0)
    def _init():
        o_ref[...] = jnp.zeros_like(o_ref)

    o_ref[...] += jnp.sum(pos_ref[...], axis=0, keepdims=True)


_pos_sum = pl.pallas_call(
    _pos_sum_body,
    grid=(SEQ // POS_BLK,),
    in_specs=[pl.BlockSpec((POS_BLK, HIDDEN), lambda i: (i, 0))],
    out_specs=pl.BlockSpec((1, HIDDEN), lambda i: (0, 0)),
    out_shape=jax.ShapeDtypeStruct((1, HIDDEN), jnp.float32),
)


def _final_body(parts_ref, pos_ref, o_ref):
    o_ref[...] = (jnp.sum(parts_ref[...], axis=0)
                  + pos_ref[0, :]) * (1.0 / SEQ)


_final = pl.pallas_call(
    _final_body,
    out_shape=jax.ShapeDtypeStruct((HIDDEN,), jnp.float32),
)


def kernel(input_ids, word_table, pos_table):
    ids = input_ids.reshape(-1).astype(jnp.int32)
    parts = _sc_partials(ids, word_table)
    pos_part = _pos_sum(pos_table)
    return _final(parts, pos_part)
